# Initial kernel scaffold; baseline (speedup 1.0000x reference)
import jax
import jax.numpy as jnp
from jax.experimental import pallas as pl


def kernel(x, edge_index, batch, W1, b1, W2, b2, bn_gamma, bn_beta, Wp, bp):
    raise NotImplementedError("write your pallas kernel here")


# speedup vs baseline: 3.3504x; 3.3504x over previous
import functools

import jax
import jax.numpy as jnp
from jax import lax
from jax.experimental import pallas as pl
from jax.experimental.pallas import tpu as pltpu
from jax.experimental.pallas import tpu_sc as plsc

N = 10000
E = 640000
D = 128
NLAYERS = 5
G = 64
NC, NS = 2, 16
NW = NC * NS
K = 128
EPW = 20480
EP = EPW * NW
NCHUNK = EPW // K
CH = 16
NGRP = NCHUNK // CH
AGG_ROWS = 10240
ZROWS = AGG_ROWS // NS

BR = 1000
RB = N // BR


def _sc_aggregate(h, srci, dsti, zrows):
    mesh = plsc.VectorSubcoreMesh(core_axis_name="c", subcore_axis_name="s")

    @functools.partial(
        pl.kernel,
        out_type=jax.ShapeDtypeStruct((NC, AGG_ROWS, D), jnp.float32),
        mesh=mesh,
        scratch_types=[
            pltpu.VMEM((CH, K), jnp.int32),
            pltpu.VMEM((CH, K), jnp.int32),
            pltpu.VMEM((K, D), jnp.float32),
            pltpu.VMEM_SHARED((AGG_ROWS, D), jnp.float32),
            pltpu.SemaphoreType.DMA,
        ],
    )
    def agg_kernel(h_hbm, srci_hbm, dsti_hbm, z_hbm, out_hbm,
                   srci_v, dsti_v, rows_v, agg_sh, sem):
        c = lax.axis_index("c")
        s = lax.axis_index("s")
        wid = c * NS + s

        pltpu.sync_copy(z_hbm, agg_sh.at[pl.ds(s * ZROWS, ZROWS)])
        plsc.subcore_barrier()

        def group(g, carry):
            pltpu.sync_copy(srci_hbm.at[wid].at[pl.ds(g * CH, CH)], srci_v)
            pltpu.sync_copy(dsti_hbm.at[wid].at[pl.ds(g * CH, CH)], dsti_v)

            def body(j, carry):
                pltpu.async_copy(h_hbm.at[srci_v.at[j]], rows_v, sem).wait()
                pltpu.sync_copy(rows_v, agg_sh.at[dsti_v.at[j]], add=True)
                return carry

            return lax.fori_loop(0, CH, body, carry)

        lax.fori_loop(0, NGRP, group, 0)
        plsc.subcore_barrier()

        pltpu.sync_copy(agg_sh.at[pl.ds(s * ZROWS, ZROWS)],
                        out_hbm.at[c].at[pl.ds(s * ZROWS, ZROWS)])

    return agg_kernel(h, srci, dsti, zrows)


def _mlp_kernel(h_ref, p0_ref, p1_ref, w1_ref, b1_ref, w2_ref, b2_ref,
                z_ref, st_ref):
    zin = h_ref[...] + p0_ref[...] + p1_ref[...]
    t = jnp.dot(zin, w1_ref[...], preferred_element_type=jnp.float32)
    t = jnp.maximum(t + b1_ref[...], 0.0)
    t = jnp.dot(t, w2_ref[...], preferred_element_type=jnp.float32)
    t = jnp.maximum(t + b2_ref[...], 0.0)
    z_ref[...] = t

    @pl.when(pl.program_id(0) == 0)
    def _():
        st_ref[...] = jnp.zeros_like(st_ref)

    su = jnp.sum(t, axis=0)
    ss = jnp.sum(t * t, axis=0)
    st_ref[...] += jnp.concatenate(
        [su[None, :], ss[None, :], jnp.zeros((6, D), jnp.float32)], axis=0)


def _mlp_pass(h, p0, p1, w1, b1, w2, b2):
    return pl.pallas_call(
        _mlp_kernel,
        grid=(RB,),
        in_specs=[
            pl.BlockSpec((BR, D), lambda r: (r, 0)),
            pl.BlockSpec((BR, D), lambda r: (r, 0)),
            pl.BlockSpec((BR, D), lambda r: (r, 0)),
            pl.BlockSpec((D, D), lambda r: (0, 0)),
            pl.BlockSpec((1, D), lambda r: (0, 0)),
            pl.BlockSpec((D, D), lambda r: (0, 0)),
            pl.BlockSpec((1, D), lambda r: (0, 0)),
        ],
        out_specs=[
            pl.BlockSpec((BR, D), lambda r: (r, 0)),
            pl.BlockSpec((8, D), lambda r: (0, 0)),
        ],
        out_shape=[
            jax.ShapeDtypeStruct((N, D), jnp.float32),
            jax.ShapeDtypeStruct((8, D), jnp.float32),
        ],
    )(h, p0, p1, w1, b1.reshape(1, D), w2, b2.reshape(1, D))


def _bn_kernel(residual, z_ref, st_ref, g_ref, be_ref, h_ref, o_ref):
    su = st_ref[0:1, :]
    ss = st_ref[1:2, :]
    mu = su * (1.0 / N)
    var = ss * (1.0 / N) - mu * mu
    zn = (z_ref[...] - mu) * lax.rsqrt(var + 1e-5) * g_ref[...] + be_ref[...]
    if residual:
        o_ref[...] = h_ref[...] + zn
    else:
        o_ref[...] = zn


def _bn_pass(z, st, gamma, beta, h, residual):
    return pl.pallas_call(
        functools.partial(_bn_kernel, residual),
        grid=(RB,),
        in_specs=[
            pl.BlockSpec((BR, D), lambda r: (r, 0)),
            pl.BlockSpec((8, D), lambda r: (0, 0)),
            pl.BlockSpec((1, D), lambda r: (0, 0)),
            pl.BlockSpec((1, D), lambda r: (0, 0)),
            pl.BlockSpec((BR, D), lambda r: (r, 0)),
        ],
        out_specs=pl.BlockSpec((BR, D), lambda r: (r, 0)),
        out_shape=jax.ShapeDtypeStruct((N, D), jnp.float32),
    )(z, st, gamma.reshape(1, D), beta.reshape(1, D), h)


def _pool_kernel(h_ref, b_ref, sum_ref, cnt_ref, max_ref):
    @pl.when(pl.program_id(0) == 0)
    def _():
        sum_ref[...] = jnp.zeros_like(sum_ref)
        cnt_ref[...] = jnp.zeros_like(cnt_ref)
        max_ref[...] = jnp.full_like(max_ref, -jnp.inf)

    hb = h_ref[...]
    bcol = b_ref[...]
    bmin = jnp.min(bcol)
    bmax = jnp.max(bcol)
    for g in range(G):
        @pl.when(jnp.logical_and(g >= bmin, g <= bmax))
        def _(g=g):
            rowmask = bcol == g
            msum = jnp.sum(jnp.where(rowmask, hb, 0.0), axis=0)
            mcnt = jnp.sum(jnp.where(rowmask, jnp.ones_like(hb), 0.0), axis=0)
            mmax = jnp.max(jnp.where(rowmask, hb, -jnp.inf), axis=0)
            sum_ref[g:g + 1, :] += msum[None, :]
            cnt_ref[g:g + 1, :] += mcnt[None, :]
            max_ref[g:g + 1, :] = jnp.maximum(max_ref[g:g + 1, :],
                                              mmax[None, :])


def _pool_pass(h, batch2):
    return pl.pallas_call(
        _pool_kernel,
        grid=(RB,),
        in_specs=[
            pl.BlockSpec((BR, D), lambda r: (r, 0)),
            pl.BlockSpec((BR, 1), lambda r: (r, 0)),
        ],
        out_specs=[
            pl.BlockSpec((G, D), lambda r: (0, 0)),
            pl.BlockSpec((G, D), lambda r: (0, 0)),
            pl.BlockSpec((G, D), lambda r: (0, 0)),
        ],
        out_shape=[
            jax.ShapeDtypeStruct((G, D), jnp.float32),
            jax.ShapeDtypeStruct((G, D), jnp.float32),
            jax.ShapeDtypeStruct((G, D), jnp.float32),
        ],
    )(h, batch2)


def _proj_kernel(sum_ref, cnt_ref, max_ref, wa_ref, wb_ref, bp_ref, o_ref):
    mean = sum_ref[...] / jnp.maximum(cnt_ref[...], 1.0)
    o_ref[...] = (jnp.dot(mean, wa_ref[...], preferred_element_type=jnp.float32)
                  + jnp.dot(max_ref[...], wb_ref[...],
                            preferred_element_type=jnp.float32)
                  + bp_ref[...])


def _proj_pass(seg_sum, seg_cnt, seg_max, Wp, bp):
    return pl.pallas_call(
        _proj_kernel,
        out_shape=jax.ShapeDtypeStruct((G, D), jnp.float32),
    )(seg_sum, seg_cnt, seg_max, Wp[:D], Wp[D:], bp.reshape(1, D))


def kernel(x, edge_index, batch, W1, b1, W2, b2, bn_gamma, bn_beta, Wp, bp):
    src = edge_index[0].astype(jnp.int32)
    dst = edge_index[1].astype(jnp.int32)
    pad = EP - E
    srci = jnp.concatenate([src, jnp.zeros((pad,), jnp.int32)]) \
        .reshape(NW, NCHUNK, K)
    dsti = jnp.concatenate([dst, jnp.full((pad,), N, jnp.int32)]) \
        .reshape(NW, NCHUNK, K)
    zrows = jnp.zeros((ZROWS, D), jnp.float32)
    batch2 = batch.astype(jnp.int32).reshape(N, 1)

    h = x
    for i in range(NLAYERS):
        parts = _sc_aggregate(h, srci, dsti, zrows)
        z, st = _mlp_pass(h, parts[0, :N], parts[1, :N],
                          W1[i], b1[i], W2[i], b2[i])
        h = _bn_pass(z, st, bn_gamma[i], bn_beta[i], h, residual=(i > 0))

    seg_sum, seg_cnt, seg_max = _pool_pass(h, batch2)
    return _proj_pass(seg_sum, seg_cnt, seg_max, Wp, bp)
